# Initial kernel scaffold; baseline (speedup 1.0000x reference)
import jax
import jax.numpy as jnp
from jax.experimental import pallas as pl


def kernel(x_0, pos, adj_0_0, lengths_0, cell_0, W_emb, b_emb, W_msg_0, b_msg_0, W_upd_0, b_upd_0, W_msg_1, b_msg_1, W_upd_1, b_upd_1, W_r1, b_r1, W_r2, b_r2):
    raise NotImplementedError("write your pallas kernel here")


# speedup vs baseline: 6.4197x; 6.4197x over previous
import functools

import jax
import jax.numpy as jnp
from jax import lax
from jax.experimental import pallas as pl
from jax.experimental.pallas import tpu as pltpu
from jax.experimental.pallas import tpu_sc as plsc

N = 10000
E = 320000
H = 128
NC, NS, L = 2, 16, 16
NW = NC * NS
EPW = E // NW
CR = 125
CE = 128
NCH = EPW // CR
NP = 10240
RPT = NP // NS
ZCH = 40
NF = H // L

BLK = 1000
GRID = N // BLK

_row = pl.BlockSpec((BLK, H), lambda i: (i, 0))
_wgt = pl.BlockSpec((H, H), lambda i: (0, 0))
_bia = pl.BlockSpec((1, H), lambda i: (0, 0))


def _dot(a, b):
    return jnp.dot(a, b, preferred_element_type=jnp.float32)


def _tc_embed_body(x_ref, we_ref, be_ref, ws_ref, wr_ref, bm_ref,
                   h_ref, a_ref, b_ref):
    h = jax.nn.silu(_dot(x_ref[...], we_ref[...]) + be_ref[...])
    h_ref[...] = h
    a_ref[...] = _dot(h, ws_ref[...])
    b_ref[...] = _dot(h, wr_ref[...]) + bm_ref[...]


_tc_embed = pl.pallas_call(
    _tc_embed_body,
    grid=(GRID,),
    in_specs=[_row, _wgt, _bia, _wgt, _wgt, _bia],
    out_specs=[_row, _row, _row],
    out_shape=[jax.ShapeDtypeStruct((N, H), jnp.float32),
               jax.ShapeDtypeStruct((NP, H), jnp.float32),
               jax.ShapeDtypeStruct((NP, H), jnp.float32)],
)


def _tc_update_body(h_ref, p0_ref, p1_ref, wua_ref, wub_ref, bu_ref,
                    ws_ref, wr_ref, bm_ref, h2_ref, a_ref, b_ref):
    h = h_ref[...]
    agg = p0_ref[...] + p1_ref[...]
    h2 = h + jax.nn.silu(_dot(h, wua_ref[...]) + _dot(agg, wub_ref[...]) + bu_ref[...])
    h2_ref[...] = h2
    a_ref[...] = _dot(h2, ws_ref[...])
    b_ref[...] = _dot(h2, wr_ref[...]) + bm_ref[...]


_tc_update = pl.pallas_call(
    _tc_update_body,
    grid=(GRID,),
    in_specs=[_row, _row, _row, _wgt, _wgt, _bia, _wgt, _wgt, _bia],
    out_specs=[_row, _row, _row],
    out_shape=[jax.ShapeDtypeStruct((N, H), jnp.float32),
               jax.ShapeDtypeStruct((NP, H), jnp.float32),
               jax.ShapeDtypeStruct((NP, H), jnp.float32)],
)


def _tc_final_body(h_ref, p0_ref, p1_ref, wua_ref, wub_ref, bu_ref,
                   w1_ref, b1_ref, w2_ref, b2_ref, out_ref):
    h = h_ref[...]
    agg = p0_ref[...] + p1_ref[...]
    h2 = h + jax.nn.silu(_dot(h, wua_ref[...]) + _dot(agg, wub_ref[...]) + bu_ref[...])
    r = jax.nn.silu(_dot(h2, w1_ref[...]) + b1_ref[...])
    out_ref[...] = _dot(r, w2_ref[...]) + b2_ref[...]


_tc_final = pl.pallas_call(
    _tc_final_body,
    grid=(GRID,),
    in_specs=[_row, _row, _row, _wgt, _wgt, _bia, _wgt, _bia, _wgt, _bia],
    out_specs=_row,
    out_shape=jax.ShapeDtypeStruct((N, H), jnp.float32),
)


@functools.lru_cache(maxsize=1)
def _sc_kernels():
    _mesh = plsc.VectorSubcoreMesh(core_axis_name="c", subcore_axis_name="s",
                                   num_cores=NC, num_subcores=NS)

    @functools.partial(
        pl.kernel,
        out_type=jax.ShapeDtypeStruct((E,), jnp.float32),
        mesh=_mesh,
        compiler_params=pltpu.CompilerParams(needs_layout_passes=False),
        scratch_types=[
            pltpu.VMEM((N,), jnp.float32),
            pltpu.VMEM((N,), jnp.float32),
            pltpu.VMEM((N,), jnp.float32),
            pltpu.VMEM((EPW,), jnp.int32),
            pltpu.VMEM((EPW,), jnp.int32),
            pltpu.VMEM((EPW,), jnp.float32),
        ],
    )
    def _sc_dpass(px_hbm, py_hbm, pz_hbm, send_hbm, recv_hbm, d_hbm,
                  px, py, pz, sidx, ridx, dv):
        wid = lax.axis_index("s") * NC + lax.axis_index("c")
        base = wid * EPW
        pltpu.sync_copy(px_hbm, px)
        pltpu.sync_copy(py_hbm, py)
        pltpu.sync_copy(pz_hbm, pz)
        pltpu.sync_copy(send_hbm.at[pl.ds(base, EPW)], sidx)
        pltpu.sync_copy(recv_hbm.at[pl.ds(base, EPW)], ridx)

        def body(i, carry):
            s = sidx[pl.ds(i * L, L)]
            r = ridx[pl.ds(i * L, L)]
            dx = plsc.load_gather(px, [s]) - plsc.load_gather(px, [r])
            dy = plsc.load_gather(py, [s]) - plsc.load_gather(py, [r])
            dz = plsc.load_gather(pz, [s]) - plsc.load_gather(pz, [r])
            sq = dx * dx + dy * dy + dz * dz + 1e-8
            xi = lax.bitcast_convert_type(sq, jnp.int32)
            y = lax.bitcast_convert_type(jnp.int32(0x5F3759DF) - (xi >> 1), jnp.float32)
            y = y * (1.5 - 0.5 * sq * y * y)
            y = y * (1.5 - 0.5 * sq * y * y)
            y = y * (1.5 - 0.5 * sq * y * y)
            dv[pl.ds(i * L, L)] = sq * y
            return carry

        lax.fori_loop(0, EPW // L, body, 0)
        pltpu.sync_copy(dv, d_hbm.at[pl.ds(base, EPW)])

    @functools.partial(
        pl.kernel,
        out_type=[jax.ShapeDtypeStruct((NP, H), jnp.float32),
                  jax.ShapeDtypeStruct((NP, H), jnp.float32)],
        mesh=_mesh,
        compiler_params=pltpu.CompilerParams(needs_layout_passes=False),
        scratch_types=[
            pltpu.VMEM((CE,), jnp.int32),
            pltpu.VMEM((CE,), jnp.int32),
            pltpu.VMEM((CE + L,), jnp.float32),
            pltpu.VMEM((H,), jnp.float32),
            pltpu.VMEM((CE, H), jnp.float32),
            pltpu.VMEM((CE, H), jnp.float32),
            pltpu.VMEM_SHARED((NP, H), jnp.float32),
            pltpu.SemaphoreType.DMA,
            pltpu.SemaphoreType.DMA,
            pltpu.SemaphoreType.DMA,
        ],
    )
    def _sc_edge(a_hbm, b_hbm, send_hbm, recv_hbm, d_hbm, wd_hbm, p0_hbm, p1_hbm,
                 schunk, rchunk, dchunk, wd, arows, brows, aggsh,
                 sem0, sem1, sem2):
        cid = lax.axis_index("c")
        sid = lax.axis_index("s")
        wid = sid * NC + cid
        pltpu.sync_copy(wd_hbm, wd)
        wdv = [wd[pl.ds(f * L, L)] for f in range(NF)]

        z16 = jnp.zeros((L,), jnp.float32)

        def zrow(i, carry):
            for f in range(NF):
                arows[i, pl.ds(f * L, L)] = z16
            return carry

        lax.fori_loop(0, ZCH, zrow, 0)
        for k in range(RPT // ZCH):
            pltpu.sync_copy(arows.at[pl.ds(0, ZCH)],
                            aggsh.at[pl.ds(sid * RPT + k * ZCH, ZCH)])
        plsc.subcore_barrier()

        def chunk(j, carry):
            cs = pltpu.async_copy(send_hbm.at[wid, j], schunk, sem2)
            cr = pltpu.async_copy(recv_hbm.at[wid, j], rchunk, sem0)
            cd = pltpu.async_copy(d_hbm.at[wid, j], dchunk.at[pl.ds(0, CE)], sem1)
            cs.wait()
            cr.wait()
            ca = pltpu.async_copy(a_hbm.at[schunk], arows, sem2)
            cb = pltpu.async_copy(b_hbm.at[rchunk], brows, sem0)
            cd.wait()
            ca.wait()
            cb.wait()

            def edge(e, c2):
                dvec = jnp.full((L,), dchunk[pl.ds(e, L)][0], jnp.float32)
                for f in range(NF):
                    sl = pl.ds(f * L, L)
                    t = arows[e, sl] + brows[e, sl] + dvec * wdv[f]
                    arows[e, sl] = t / (1.0 + jnp.exp(-t))
                return c2

            lax.fori_loop(0, CE, edge, 0)
            pltpu.sync_copy(arows, aggsh.at[rchunk], add=True)
            return carry

        lax.fori_loop(0, NCH, chunk, 0)
        plsc.subcore_barrier()

        @pl.when(cid == 0)
        def _():
            pltpu.sync_copy(aggsh.at[pl.ds(sid * RPT, RPT)],
                            p0_hbm.at[pl.ds(sid * RPT, RPT)])

        @pl.when(cid == 1)
        def _():
            pltpu.sync_copy(aggsh.at[pl.ds(sid * RPT, RPT)],
                            p1_hbm.at[pl.ds(sid * RPT, RPT)])

    return _sc_dpass, _sc_edge


def kernel(x_0, pos, adj_0_0, lengths_0, cell_0, W_emb, b_emb,
           W_msg_0, b_msg_0, W_upd_0, b_upd_0, W_msg_1, b_msg_1,
           W_upd_1, b_upd_1, W_r1, b_r1, W_r2, b_r2):
    send = adj_0_0[0]
    recv = adj_0_0[1]
    pad3 = ((0, 0), (0, 0), (0, CE - CR))
    send_pad = jnp.pad(send.reshape(NW, NCH, CR), pad3)
    recv_pad = jnp.pad(recv.reshape(NW, NCH, CR), pad3, constant_values=N)
    px, py, pz = pos[:, 0], pos[:, 1], pos[:, 2]

    Ws0, Wr0 = W_msg_0[:H], W_msg_0[H:2 * H]
    wd0 = W_msg_0[2 * H] + W_msg_0[2 * H + 3] + W_msg_0[2 * H + 4]
    Ws1, Wr1p = W_msg_1[:H], W_msg_1[H:2 * H]
    wd1 = W_msg_1[2 * H] + W_msg_1[2 * H + 3] + W_msg_1[2 * H + 4]

    h0, A0, B0 = _tc_embed(x_0, W_emb, b_emb.reshape(1, H),
                           Ws0, Wr0, b_msg_0.reshape(1, H))
    _sc_dpass, _sc_edge = _sc_kernels()
    d = _sc_dpass(px, py, pz, send, recv)
    d_pad = jnp.pad(d.reshape(NW, NCH, CR), pad3)
    P0a, P0b = _sc_edge(A0, B0, send_pad, recv_pad, d_pad, wd0)
    h1, A1, B1 = _tc_update(h0, P0a, P0b, W_upd_0[:H], W_upd_0[H:],
                            b_upd_0.reshape(1, H), Ws1, Wr1p,
                            b_msg_1.reshape(1, H))
    P1a, P1b = _sc_edge(A1, B1, send_pad, recv_pad, d_pad, wd1)
    out = _tc_final(h1, P1a, P1b, W_upd_1[:H], W_upd_1[H:],
                    b_upd_1.reshape(1, H), W_r1, b_r1.reshape(1, H),
                    W_r2, b_r2.reshape(1, H))
    return out


# speedup vs baseline: 9.3962x; 1.4637x over previous
import functools

import jax
import jax.numpy as jnp
from jax import lax
from jax.experimental import pallas as pl
from jax.experimental.pallas import tpu as pltpu
from jax.experimental.pallas import tpu_sc as plsc

N = 10000
E = 320000
H = 128
NC, NS, L = 2, 16, 16
NW = NC * NS
EPW = E // NW
CE = 80
NCH = EPW // CE
NP = 10240
RPT = NP // NS
ZCH = 40
NF = H // L

BLK = 1000
GRID = N // BLK

_row = pl.BlockSpec((BLK, H), lambda i: (i, 0))
_wgt = pl.BlockSpec((H, H), lambda i: (0, 0))
_bia = pl.BlockSpec((1, H), lambda i: (0, 0))


def _dot(a, b):
    return jnp.dot(a, b, preferred_element_type=jnp.float32)


def _tc_embed_body(x_ref, we_ref, be_ref, ws_ref, wr_ref, bm_ref,
                   h_ref, a_ref, b_ref):
    h = jax.nn.silu(_dot(x_ref[...], we_ref[...]) + be_ref[...])
    h_ref[...] = h
    a_ref[...] = _dot(h, ws_ref[...])
    b_ref[...] = _dot(h, wr_ref[...]) + bm_ref[...]


_tc_embed = pl.pallas_call(
    _tc_embed_body,
    grid=(GRID,),
    in_specs=[_row, _wgt, _bia, _wgt, _wgt, _bia],
    out_specs=[_row, _row, _row],
    out_shape=[jax.ShapeDtypeStruct((N, H), jnp.float32),
               jax.ShapeDtypeStruct((NP, H), jnp.float32),
               jax.ShapeDtypeStruct((NP, H), jnp.float32)],
)


def _tc_update_body(h_ref, p0_ref, p1_ref, wua_ref, wub_ref, bu_ref,
                    ws_ref, wr_ref, bm_ref, h2_ref, a_ref, b_ref):
    h = h_ref[...]
    agg = p0_ref[...] + p1_ref[...]
    h2 = h + jax.nn.silu(_dot(h, wua_ref[...]) + _dot(agg, wub_ref[...]) + bu_ref[...])
    h2_ref[...] = h2
    a_ref[...] = _dot(h2, ws_ref[...])
    b_ref[...] = _dot(h2, wr_ref[...]) + bm_ref[...]


_tc_update = pl.pallas_call(
    _tc_update_body,
    grid=(GRID,),
    in_specs=[_row, _row, _row, _wgt, _wgt, _bia, _wgt, _wgt, _bia],
    out_specs=[_row, _row, _row],
    out_shape=[jax.ShapeDtypeStruct((N, H), jnp.float32),
               jax.ShapeDtypeStruct((NP, H), jnp.float32),
               jax.ShapeDtypeStruct((NP, H), jnp.float32)],
)


def _tc_final_body(h_ref, p0_ref, p1_ref, wua_ref, wub_ref, bu_ref,
                   w1_ref, b1_ref, w2_ref, b2_ref, out_ref):
    h = h_ref[...]
    agg = p0_ref[...] + p1_ref[...]
    h2 = h + jax.nn.silu(_dot(h, wua_ref[...]) + _dot(agg, wub_ref[...]) + bu_ref[...])
    r = jax.nn.silu(_dot(h2, w1_ref[...]) + b1_ref[...])
    out_ref[...] = _dot(r, w2_ref[...]) + b2_ref[...]


_tc_final = pl.pallas_call(
    _tc_final_body,
    grid=(GRID,),
    in_specs=[_row, _row, _row, _wgt, _wgt, _bia, _wgt, _bia, _wgt, _bia],
    out_specs=_row,
    out_shape=jax.ShapeDtypeStruct((N, H), jnp.float32),
)


@functools.lru_cache(maxsize=1)
def _sc_kernels():
    _mesh = plsc.VectorSubcoreMesh(core_axis_name="c", subcore_axis_name="s",
                                   num_cores=NC, num_subcores=NS)

    @functools.partial(
        pl.kernel,
        out_type=jax.ShapeDtypeStruct((E,), jnp.float32),
        mesh=_mesh,
        compiler_params=pltpu.CompilerParams(needs_layout_passes=False),
        scratch_types=[
            pltpu.VMEM((N,), jnp.float32),
            pltpu.VMEM((N,), jnp.float32),
            pltpu.VMEM((N,), jnp.float32),
            pltpu.VMEM((EPW,), jnp.int32),
            pltpu.VMEM((EPW,), jnp.int32),
            pltpu.VMEM((EPW,), jnp.float32),
        ],
    )
    def _sc_dpass(px_hbm, py_hbm, pz_hbm, send_hbm, recv_hbm, d_hbm,
                  px, py, pz, sidx, ridx, dv):
        wid = lax.axis_index("s") * NC + lax.axis_index("c")
        base = wid * EPW
        pltpu.sync_copy(px_hbm, px)
        pltpu.sync_copy(py_hbm, py)
        pltpu.sync_copy(pz_hbm, pz)
        pltpu.sync_copy(send_hbm.at[pl.ds(base, EPW)], sidx)
        pltpu.sync_copy(recv_hbm.at[pl.ds(base, EPW)], ridx)

        def body(i, carry):
            s = sidx[pl.ds(i * L, L)]
            r = ridx[pl.ds(i * L, L)]
            dx = plsc.load_gather(px, [s]) - plsc.load_gather(px, [r])
            dy = plsc.load_gather(py, [s]) - plsc.load_gather(py, [r])
            dz = plsc.load_gather(pz, [s]) - plsc.load_gather(pz, [r])
            sq = dx * dx + dy * dy + dz * dz + 1e-8
            xi = lax.bitcast_convert_type(sq, jnp.int32)
            y = lax.bitcast_convert_type(jnp.int32(0x5F3759DF) - (xi >> 1), jnp.float32)
            y = y * (1.5 - 0.5 * sq * y * y)
            y = y * (1.5 - 0.5 * sq * y * y)
            y = y * (1.5 - 0.5 * sq * y * y)
            dv[pl.ds(i * L, L)] = sq * y
            return carry

        lax.fori_loop(0, EPW // L, body, 0)
        pltpu.sync_copy(dv, d_hbm.at[pl.ds(base, EPW)])

    @functools.partial(
        pl.kernel,
        out_type=[jax.ShapeDtypeStruct((NP, H), jnp.float32),
                  jax.ShapeDtypeStruct((NP, H), jnp.float32)],
        mesh=_mesh,
        compiler_params=pltpu.CompilerParams(needs_layout_passes=False),
        scratch_types=[
            pltpu.VMEM((2, CE), jnp.int32),
            pltpu.VMEM((2, CE), jnp.int32),
            pltpu.VMEM((2, CE + L), jnp.float32),
            pltpu.VMEM((H,), jnp.float32),
            pltpu.VMEM((2, CE, H), jnp.float32),
            pltpu.VMEM((2, CE, H), jnp.float32),
            pltpu.VMEM_SHARED((NP, H), jnp.float32),
            [pltpu.SemaphoreType.DMA] * 2,
            [pltpu.SemaphoreType.DMA] * 2,
            [pltpu.SemaphoreType.DMA] * 2,
        ],
    )
    def _sc_edge(a_hbm, b_hbm, send_hbm, recv_hbm, d_hbm, wd_hbm, p0_hbm, p1_hbm,
                 schunk, rchunk, dchunk, wd, arows, brows, aggsh,
                 s_idx, s_rows, s_sc):
        cid = lax.axis_index("c")
        sid = lax.axis_index("s")
        wid = sid * NC + cid
        pltpu.sync_copy(wd_hbm, wd)
        wdv = [wd[pl.ds(f * L, L)] for f in range(NF)]

        z16 = jnp.zeros((L,), jnp.float32)

        def zrow(i, carry):
            for f in range(NF):
                arows[0, i, pl.ds(f * L, L)] = z16
            return carry

        lax.fori_loop(0, ZCH, zrow, 0)
        for k in range(RPT // ZCH):
            pltpu.sync_copy(arows.at[0, pl.ds(0, ZCH)],
                            aggsh.at[pl.ds(sid * RPT + k * ZCH, ZCH)])
        plsc.subcore_barrier()

        def issue_idx(c, p):
            pltpu.async_copy(send_hbm.at[wid, c], schunk.at[p], s_idx[p])
            pltpu.async_copy(recv_hbm.at[wid, c], rchunk.at[p], s_idx[p])
            pltpu.async_copy(d_hbm.at[wid, c], dchunk.at[p, pl.ds(0, CE)], s_idx[p])

        def wait_idx(p):
            pltpu.make_async_copy(send_hbm.at[wid, 0], schunk.at[p], s_idx[p]).wait()
            pltpu.make_async_copy(recv_hbm.at[wid, 0], rchunk.at[p], s_idx[p]).wait()
            pltpu.make_async_copy(d_hbm.at[wid, 0], dchunk.at[p, pl.ds(0, CE)], s_idx[p]).wait()

        def issue_gather(p):
            pltpu.async_copy(a_hbm.at[schunk.at[p]], arows.at[p], s_rows[p])
            pltpu.async_copy(b_hbm.at[rchunk.at[p]], brows.at[p], s_rows[p])

        def wait_gather(p):
            pltpu.make_async_copy(a_hbm.at[schunk.at[p]], arows.at[p], s_rows[p]).wait()
            pltpu.make_async_copy(b_hbm.at[rchunk.at[p]], brows.at[p], s_rows[p]).wait()

        def wait_scatter(p):
            pltpu.make_async_copy(arows.at[p], aggsh.at[rchunk.at[p]], s_sc[p]).wait()

        def process(c, p, gather_next, issue_next, wait_prev_scatter):
            q = 1 - p
            if gather_next:
                if wait_prev_scatter:
                    wait_scatter(q)
                wait_idx(q)
                issue_gather(q)
            wait_gather(p)

            def edge(e, c2):
                dvec = jnp.full((L,), dchunk[p, pl.ds(e, L)][0], jnp.float32)
                for f in range(NF):
                    sl = pl.ds(f * L, L)
                    t = arows[p, e, sl] + brows[p, e, sl] + dvec * wdv[f]
                    arows[p, e, sl] = t / (1.0 + jnp.exp(-t))
                return c2

            lax.fori_loop(0, CE, edge, 0)
            pltpu.async_copy(arows.at[p], aggsh.at[rchunk.at[p]], s_sc[p], add=True)
            if issue_next:
                issue_idx(c + 2, p)

        issue_idx(0, 0)
        wait_idx(0)
        issue_gather(0)
        issue_idx(1, 1)
        process(0, 0, True, True, False)

        def pair(g, carry):
            c = 2 * g + 1
            process(c, 1, True, True, True)
            process(c + 1, 0, True, True, True)
            return carry

        lax.fori_loop(0, (NCH - 3) // 2, pair, 0)
        process(NCH - 2, 1, True, False, True)
        process(NCH - 1, 0, False, False, False)
        wait_scatter(1)
        wait_scatter(0)
        plsc.subcore_barrier()

        @pl.when(cid == 0)
        def _():
            pltpu.sync_copy(aggsh.at[pl.ds(sid * RPT, RPT)],
                            p0_hbm.at[pl.ds(sid * RPT, RPT)])

        @pl.when(cid == 1)
        def _():
            pltpu.sync_copy(aggsh.at[pl.ds(sid * RPT, RPT)],
                            p1_hbm.at[pl.ds(sid * RPT, RPT)])

    return _sc_dpass, _sc_edge


def kernel(x_0, pos, adj_0_0, lengths_0, cell_0, W_emb, b_emb,
           W_msg_0, b_msg_0, W_upd_0, b_upd_0, W_msg_1, b_msg_1,
           W_upd_1, b_upd_1, W_r1, b_r1, W_r2, b_r2):
    send = adj_0_0[0]
    recv = adj_0_0[1]
    send_pad = send.reshape(NW, NCH, CE)
    recv_pad = recv.reshape(NW, NCH, CE)
    px, py, pz = pos[:, 0], pos[:, 1], pos[:, 2]

    Ws0, Wr0 = W_msg_0[:H], W_msg_0[H:2 * H]
    wd0 = W_msg_0[2 * H] + W_msg_0[2 * H + 3] + W_msg_0[2 * H + 4]
    Ws1, Wr1p = W_msg_1[:H], W_msg_1[H:2 * H]
    wd1 = W_msg_1[2 * H] + W_msg_1[2 * H + 3] + W_msg_1[2 * H + 4]

    h0, A0, B0 = _tc_embed(x_0, W_emb, b_emb.reshape(1, H),
                           Ws0, Wr0, b_msg_0.reshape(1, H))
    _sc_dpass, _sc_edge = _sc_kernels()
    d = _sc_dpass(px, py, pz, send, recv)
    d_pad = d.reshape(NW, NCH, CE)
    P0a, P0b = _sc_edge(A0, B0, send_pad, recv_pad, d_pad, wd0)
    h1, A1, B1 = _tc_update(h0, P0a, P0b, W_upd_0[:H], W_upd_0[H:],
                            b_upd_0.reshape(1, H), Ws1, Wr1p,
                            b_msg_1.reshape(1, H))
    P1a, P1b = _sc_edge(A1, B1, send_pad, recv_pad, d_pad, wd1)
    out = _tc_final(h1, P1a, P1b, W_upd_1[:H], W_upd_1[H:],
                    b_upd_1.reshape(1, H), W_r1, b_r1.reshape(1, H),
                    W_r2, b_r2.reshape(1, H))
    return out


# speedup vs baseline: 11.4947x; 1.2233x over previous
import functools

import jax
import jax.numpy as jnp
from jax import lax
from jax.experimental import pallas as pl
from jax.experimental.pallas import tpu as pltpu
from jax.experimental.pallas import tpu_sc as plsc

N = 10000
E = 320000
H = 128
NC, NS, L = 2, 16, 16
NW = NC * NS
EPW = E // NW
CE = 80
NCH = EPW // CE
NP = 10240
RPT = NP // NS
ZCH = 40
NF = H // L

BLK = 2000
GRID = N // BLK

_row = pl.BlockSpec((BLK, H), lambda i: (i, 0))
_wgt = pl.BlockSpec((H, H), lambda i: (0, 0))
_bia = pl.BlockSpec((1, H), lambda i: (0, 0))


def _dot(a, b):
    return jnp.dot(a, b, preferred_element_type=jnp.float32)


def _tc_embed_body(x_ref, we_ref, be_ref, ws_ref, wr_ref, bm_ref,
                   h_ref, a_ref, b_ref):
    h = jax.nn.silu(_dot(x_ref[...], we_ref[...]) + be_ref[...])
    h_ref[...] = h
    a_ref[...] = _dot(h, ws_ref[...])
    b_ref[...] = _dot(h, wr_ref[...]) + bm_ref[...]


_tc_embed = pl.pallas_call(
    _tc_embed_body,
    grid=(GRID,),
    in_specs=[_row, _wgt, _bia, _wgt, _wgt, _bia],
    out_specs=[_row, _row, _row],
    out_shape=[jax.ShapeDtypeStruct((N, H), jnp.float32),
               jax.ShapeDtypeStruct((NP, H), jnp.float32),
               jax.ShapeDtypeStruct((NP, H), jnp.float32)],
)


def _tc_update_body(h_ref, p0_ref, p1_ref, wua_ref, wub_ref, bu_ref,
                    ws_ref, wr_ref, bm_ref, h2_ref, a_ref, b_ref):
    h = h_ref[...]
    agg = p0_ref[...] + p1_ref[...]
    h2 = h + jax.nn.silu(_dot(h, wua_ref[...]) + _dot(agg, wub_ref[...]) + bu_ref[...])
    h2_ref[...] = h2
    a_ref[...] = _dot(h2, ws_ref[...])
    b_ref[...] = _dot(h2, wr_ref[...]) + bm_ref[...]


_tc_update = pl.pallas_call(
    _tc_update_body,
    grid=(GRID,),
    in_specs=[_row, _row, _row, _wgt, _wgt, _bia, _wgt, _wgt, _bia],
    out_specs=[_row, _row, _row],
    out_shape=[jax.ShapeDtypeStruct((N, H), jnp.float32),
               jax.ShapeDtypeStruct((NP, H), jnp.float32),
               jax.ShapeDtypeStruct((NP, H), jnp.float32)],
)


def _tc_final_body(h_ref, p0_ref, p1_ref, wua_ref, wub_ref, bu_ref,
                   w1_ref, b1_ref, w2_ref, b2_ref, out_ref):
    h = h_ref[...]
    agg = p0_ref[...] + p1_ref[...]
    h2 = h + jax.nn.silu(_dot(h, wua_ref[...]) + _dot(agg, wub_ref[...]) + bu_ref[...])
    r = jax.nn.silu(_dot(h2, w1_ref[...]) + b1_ref[...])
    out_ref[...] = _dot(r, w2_ref[...]) + b2_ref[...]


_tc_final = pl.pallas_call(
    _tc_final_body,
    grid=(GRID,),
    in_specs=[_row, _row, _row, _wgt, _wgt, _bia, _wgt, _bia, _wgt, _bia],
    out_specs=_row,
    out_shape=jax.ShapeDtypeStruct((N, H), jnp.float32),
)


@functools.lru_cache(maxsize=1)
def _sc_kernels():
    _mesh = plsc.VectorSubcoreMesh(core_axis_name="c", subcore_axis_name="s",
                                   num_cores=NC, num_subcores=NS)

    @functools.partial(
        pl.kernel,
        out_type=jax.ShapeDtypeStruct((E,), jnp.float32),
        mesh=_mesh,
        compiler_params=pltpu.CompilerParams(needs_layout_passes=False),
        scratch_types=[
            pltpu.VMEM((N,), jnp.float32),
            pltpu.VMEM((N,), jnp.float32),
            pltpu.VMEM((N,), jnp.float32),
            pltpu.VMEM((EPW,), jnp.int32),
            pltpu.VMEM((EPW,), jnp.int32),
            pltpu.VMEM((EPW,), jnp.float32),
        ],
    )
    def _sc_dpass(px_hbm, py_hbm, pz_hbm, send_hbm, recv_hbm, d_hbm,
                  px, py, pz, sidx, ridx, dv):
        wid = lax.axis_index("s") * NC + lax.axis_index("c")
        base = wid * EPW
        pltpu.sync_copy(px_hbm, px)
        pltpu.sync_copy(py_hbm, py)
        pltpu.sync_copy(pz_hbm, pz)
        pltpu.sync_copy(send_hbm.at[pl.ds(base, EPW)], sidx)
        pltpu.sync_copy(recv_hbm.at[pl.ds(base, EPW)], ridx)

        @plsc.parallel_loop(0, EPW // L, unroll=2)
        def body(i):
            s = sidx[pl.ds(i * L, L)]
            r = ridx[pl.ds(i * L, L)]
            dx = plsc.load_gather(px, [s]) - plsc.load_gather(px, [r])
            dy = plsc.load_gather(py, [s]) - plsc.load_gather(py, [r])
            dz = plsc.load_gather(pz, [s]) - plsc.load_gather(pz, [r])
            sq = dx * dx + dy * dy + dz * dz + 1e-8
            xi = lax.bitcast_convert_type(sq, jnp.int32)
            y = lax.bitcast_convert_type(jnp.int32(0x5F3759DF) - (xi >> 1), jnp.float32)
            y = y * (1.5 - 0.5 * sq * y * y)
            y = y * (1.5 - 0.5 * sq * y * y)
            y = y * (1.5 - 0.5 * sq * y * y)
            dv[pl.ds(i * L, L)] = sq * y

        pltpu.sync_copy(dv, d_hbm.at[pl.ds(base, EPW)])

    @functools.partial(
        pl.kernel,
        out_type=[jax.ShapeDtypeStruct((NP, H), jnp.float32),
                  jax.ShapeDtypeStruct((NP, H), jnp.float32)],
        mesh=_mesh,
        compiler_params=pltpu.CompilerParams(needs_layout_passes=False),
        scratch_types=[
            pltpu.VMEM((2, CE), jnp.int32),
            pltpu.VMEM((2, CE), jnp.int32),
            pltpu.VMEM((2, CE + L), jnp.float32),
            pltpu.VMEM((H,), jnp.float32),
            pltpu.VMEM((2, CE, H), jnp.float32),
            pltpu.VMEM((2, CE, H), jnp.float32),
            pltpu.VMEM_SHARED((NP, H), jnp.float32),
            [pltpu.SemaphoreType.DMA] * 2,
            [pltpu.SemaphoreType.DMA] * 2,
            [pltpu.SemaphoreType.DMA] * 2,
        ],
    )
    def _sc_edge(a_hbm, b_hbm, send_hbm, recv_hbm, d_hbm, wd_hbm, p0_hbm, p1_hbm,
                 schunk, rchunk, dchunk, wd, arows, brows, aggsh,
                 s_idx, s_rows, s_sc):
        cid = lax.axis_index("c")
        sid = lax.axis_index("s")
        wid = sid * NC + cid
        def issue_idx(c, p):
            pltpu.async_copy(send_hbm.at[wid, c], schunk.at[p], s_idx[p])
            pltpu.async_copy(recv_hbm.at[wid, c], rchunk.at[p], s_idx[p])
            pltpu.async_copy(d_hbm.at[wid, c], dchunk.at[p, pl.ds(0, CE)], s_idx[p])

        def wait_idx(p):
            pltpu.make_async_copy(send_hbm.at[wid, 0], schunk.at[p], s_idx[p]).wait()
            pltpu.make_async_copy(recv_hbm.at[wid, 0], rchunk.at[p], s_idx[p]).wait()
            pltpu.make_async_copy(d_hbm.at[wid, 0], dchunk.at[p, pl.ds(0, CE)], s_idx[p]).wait()

        def issue_gather(p):
            pltpu.async_copy(a_hbm.at[schunk.at[p]], arows.at[p], s_rows[p])
            pltpu.async_copy(b_hbm.at[rchunk.at[p]], brows.at[p], s_rows[p])

        def wait_gather(p):
            pltpu.make_async_copy(a_hbm.at[schunk.at[p]], arows.at[p], s_rows[p]).wait()
            pltpu.make_async_copy(b_hbm.at[rchunk.at[p]], brows.at[p], s_rows[p]).wait()

        def wait_scatter(p):
            pltpu.make_async_copy(arows.at[p], aggsh.at[rchunk.at[p]], s_sc[p]).wait()

        def process(c, p, gather_next, issue_next, wait_prev_scatter):
            q = 1 - p
            if gather_next:
                if wait_prev_scatter:
                    wait_scatter(q)
                wait_idx(q)
                issue_gather(q)
            wait_gather(p)

            @plsc.parallel_loop(0, CE, unroll=2)
            def edge(e):
                dvec = jnp.full((L,), dchunk[p, pl.ds(e, L)][0], jnp.float32)
                for f in range(NF):
                    sl = pl.ds(f * L, L)
                    t = arows[p, e, sl] + brows[p, e, sl] + dvec * wdv[f]
                    arows[p, e, sl] = t / (1.0 + jnp.exp(-t))

            pltpu.async_copy(arows.at[p], aggsh.at[rchunk.at[p]], s_sc[p], add=True)
            if issue_next:
                issue_idx(c + 2, p)

        issue_idx(0, 0)
        issue_idx(1, 1)
        pltpu.sync_copy(wd_hbm, wd)
        wdv = [wd[pl.ds(f * L, L)] for f in range(NF)]

        z16 = jnp.zeros((L,), jnp.float32)

        def zrow(i, carry):
            for f in range(NF):
                arows[0, i, pl.ds(f * L, L)] = z16
            return carry

        lax.fori_loop(0, ZCH, zrow, 0)
        for k in range(RPT // ZCH):
            pltpu.sync_copy(arows.at[0, pl.ds(0, ZCH)],
                            aggsh.at[pl.ds(sid * RPT + k * ZCH, ZCH)])
        plsc.subcore_barrier()

        wait_idx(0)
        issue_gather(0)
        process(0, 0, True, True, False)

        def pair(g, carry):
            c = 2 * g + 1
            process(c, 1, True, True, True)
            process(c + 1, 0, True, True, True)
            return carry

        lax.fori_loop(0, (NCH - 3) // 2, pair, 0)
        process(NCH - 2, 1, True, False, True)
        process(NCH - 1, 0, False, False, False)
        wait_scatter(1)
        wait_scatter(0)
        plsc.subcore_barrier()

        @pl.when(cid == 0)
        def _():
            pltpu.sync_copy(aggsh.at[pl.ds(sid * RPT, RPT)],
                            p0_hbm.at[pl.ds(sid * RPT, RPT)])

        @pl.when(cid == 1)
        def _():
            pltpu.sync_copy(aggsh.at[pl.ds(sid * RPT, RPT)],
                            p1_hbm.at[pl.ds(sid * RPT, RPT)])

    return _sc_dpass, _sc_edge


def kernel(x_0, pos, adj_0_0, lengths_0, cell_0, W_emb, b_emb,
           W_msg_0, b_msg_0, W_upd_0, b_upd_0, W_msg_1, b_msg_1,
           W_upd_1, b_upd_1, W_r1, b_r1, W_r2, b_r2):
    send = adj_0_0[0]
    recv = adj_0_0[1]
    send_pad = send.reshape(NW, NCH, CE)
    recv_pad = recv.reshape(NW, NCH, CE)
    px, py, pz = pos[:, 0], pos[:, 1], pos[:, 2]

    Ws0, Wr0 = W_msg_0[:H], W_msg_0[H:2 * H]
    wd0 = W_msg_0[2 * H] + W_msg_0[2 * H + 3] + W_msg_0[2 * H + 4]
    Ws1, Wr1p = W_msg_1[:H], W_msg_1[H:2 * H]
    wd1 = W_msg_1[2 * H] + W_msg_1[2 * H + 3] + W_msg_1[2 * H + 4]

    h0, A0, B0 = _tc_embed(x_0, W_emb, b_emb.reshape(1, H),
                           Ws0, Wr0, b_msg_0.reshape(1, H))
    _sc_dpass, _sc_edge = _sc_kernels()
    d = _sc_dpass(px, py, pz, send, recv)
    d_pad = d.reshape(NW, NCH, CE)
    P0a, P0b = _sc_edge(A0, B0, send_pad, recv_pad, d_pad, wd0)
    h1, A1, B1 = _tc_update(h0, P0a, P0b, W_upd_0[:H], W_upd_0[H:],
                            b_upd_0.reshape(1, H), Ws1, Wr1p,
                            b_msg_1.reshape(1, H))
    P1a, P1b = _sc_edge(A1, B1, send_pad, recv_pad, d_pad, wd1)
    out = _tc_final(h1, P1a, P1b, W_upd_1[:H], W_upd_1[H:],
                    b_upd_1.reshape(1, H), W_r1, b_r1.reshape(1, H),
                    W_r2, b_r2.reshape(1, H))
    return out
